# baseline (speedup 1.0000x reference)
import jax
import jax.numpy as jnp
from jax import lax
from jax.experimental import pallas as pl
from jax.experimental.pallas import tpu as pltpu
from jax.experimental.pallas import tpu_sc as plsc

B = 16384
D = 128
NC = 2
NS = 16
NW = NC * NS
PER_W = B // NW
CHUNK = 128
NCH = PER_W // CHUNK
NLANE = 16
NACC = D // NLANE
NGRP = CHUNK // NLANE
NUM_ROWS8 = 12500
NBUF = 3


def _sc_body(idx_h, ue_h, be_h, bias_h,
             partials_o, ubg_o, bbg_o,
             idx_v, idx8_v, ubuf, bbuf,
             ubias8, bbias8, ubias_f, bbias_f, acc_v,
             emb_sem, bias_sem, out_sem):
    wid = lax.axis_index("s") * NC + lax.axis_index("c")
    base = wid * PER_W

    pltpu.sync_copy(idx_h.at[wid], idx_v)

    for r in range(2 * NCH):
        off = 0 if r < NCH else NUM_ROWS8
        for g in range(NGRP):
            v = idx_v[r, pl.ds(g * NLANE, NLANE)]
            idx8_v[r, pl.ds(g * NLANE, NLANE)] = (
                lax.shift_right_logical(v, 3) + off)

    def issue(j, p):
        return (
            pltpu.async_copy(ue_h.at[idx_v.at[j]], ubuf.at[p], emb_sem.at[p]),
            pltpu.async_copy(be_h.at[idx_v.at[NCH + j]], bbuf.at[p],
                             emb_sem.at[p]),
            pltpu.async_copy(bias_h.at[idx8_v.at[j]], ubias8.at[p],
                             bias_sem.at[p]),
            pltpu.async_copy(bias_h.at[idx8_v.at[NCH + j]], bbias8.at[p],
                             bias_sem.at[p]),
        )

    accs = [jnp.zeros((NLANE,), jnp.float32) for _ in range(2 * NACC)]
    outcps = []
    pend = [issue(0, 0), issue(1, 1), issue(2, 2)]
    for j in range(NCH):
        p = j % NBUF
        cps = pend[0]
        cps[2].wait()
        cps[3].wait()
        for g in range(NGRP):
            pos = lax.iota(jnp.int32, NLANE) + (g * NLANE)
            ui = idx_v[j, pl.ds(g * NLANE, NLANE)]
            bi = idx_v[NCH + j, pl.ds(g * NLANE, NLANE)]
            uv = plsc.load_gather(ubias8.at[p], [pos, jnp.bitwise_and(ui, 7)])
            bv = plsc.load_gather(bbias8.at[p], [pos, jnp.bitwise_and(bi, 7)])
            ubias_f[p, pl.ds(g * NLANE, NLANE)] = uv
            bbias_f[p, pl.ds(g * NLANE, NLANE)] = bv
        off = base + j * CHUNK
        outcps.append(pltpu.async_copy(
            ubias_f.at[p], ubg_o.at[pl.ds(off, CHUNK)], out_sem))
        outcps.append(pltpu.async_copy(
            bbias_f.at[p], bbg_o.at[pl.ds(off, CHUNK)], out_sem))

        cps[0].wait()
        cps[1].wait()
        ub_j, bb_j = ubuf.at[p], bbuf.at[p]

        def row_body(r, accs, ub_j=ub_j, bb_j=bb_j):
            out = []
            for k in range(2):
                for c in range(NACC):
                    u = ub_j[2 * r + k, pl.ds(c * NLANE, NLANE)]
                    b = bb_j[2 * r + k, pl.ds(c * NLANE, NLANE)]
                    out.append(accs[k * NACC + c] + u * b)
            return tuple(out)

        accs = list(lax.fori_loop(0, CHUNK // 2, row_body, tuple(accs)))
        pend.pop(0)
        if j + NBUF < NCH:
            pend.append(issue(j + NBUF, p))

    total = accs[0]
    for c in range(1, 2 * NACC):
        total = total + accs[c]
    acc_v[...] = total
    for cp in outcps:
        cp.wait()
    pltpu.sync_copy(acc_v, partials_o.at[pl.ds(wid * NLANE, NLANE)])


def _tc_body(p_ref, u_ref, b_ref, o_ref):
    total = jnp.sum(p_ref[...])
    o_ref[...] = jax.nn.sigmoid(total + u_ref[...] + b_ref[...])


def kernel(inputs, user_embedding, user_bias, book_embedding, book_bias):
    uidx = inputs[:, 0].reshape(NW, NCH, CHUNK)
    bidx = inputs[:, 1].reshape(NW, NCH, CHUNK)
    idx_all = jnp.concatenate([uidx, bidx], axis=1)
    bias_all = jnp.concatenate(
        [user_bias.T, book_bias.T], axis=0).reshape(2 * NUM_ROWS8, 8)

    mesh = plsc.VectorSubcoreMesh(
        core_axis_name="c", subcore_axis_name="s",
        num_cores=NC, num_subcores=NS)

    sc = pl.kernel(
        _sc_body,
        out_type=(
            jax.ShapeDtypeStruct((NW * NLANE,), jnp.float32),
            jax.ShapeDtypeStruct((B,), jnp.float32),
            jax.ShapeDtypeStruct((B,), jnp.float32),
        ),
        mesh=mesh,
        scratch_types=[
            pltpu.VMEM((2 * NCH, CHUNK), jnp.int32),
            pltpu.VMEM((2 * NCH, CHUNK), jnp.int32),
            pltpu.VMEM((NBUF, CHUNK, D), jnp.float32),
            pltpu.VMEM((NBUF, CHUNK, D), jnp.float32),
            pltpu.VMEM((NBUF, CHUNK, 8), jnp.float32),
            pltpu.VMEM((NBUF, CHUNK, 8), jnp.float32),
            pltpu.VMEM((NBUF, CHUNK), jnp.float32),
            pltpu.VMEM((NBUF, CHUNK), jnp.float32),
            pltpu.VMEM((NLANE,), jnp.float32),
            pltpu.SemaphoreType.DMA((NBUF,)),
            pltpu.SemaphoreType.DMA((NBUF,)),
            pltpu.SemaphoreType.DMA,
        ],
        compiler_params=pltpu.CompilerParams(
            use_tc_tiling_on_sc=False, needs_layout_passes=False),
    )
    partials, ubg, bbg = sc(idx_all, user_embedding, book_embedding, bias_all)

    out = pl.pallas_call(
        _tc_body,
        out_shape=jax.ShapeDtypeStruct((B // D, D), jnp.float32),
    )(partials.reshape(NW * NLANE // D, D),
      ubg.reshape(B // D, D), bbg.reshape(B // D, D))
    return out.reshape(B, 1)


# speedup vs baseline: 1.0002x; 1.0002x over previous
import jax
import jax.numpy as jnp
from jax import lax
from jax.experimental import pallas as pl
from jax.experimental.pallas import tpu as pltpu
from jax.experimental.pallas import tpu_sc as plsc

B = 16384
D = 128
NC = 2
NS = 16
NW = NC * NS
PER_W = B // NW
CHUNK = 128
NCH = PER_W // CHUNK
NLANE = 16
NACC = D // NLANE
NGRP = CHUNK // NLANE
NUM_ROWS8 = 12500
NBUF = 3


def _sc_body(idx_h, ue_h, be_h, bias_h,
             partials_o, sbg_o,
             idx_v, idx8_v, ubuf, bbuf,
             ubias8, bbias8, sbias_f, acc_v,
             emb_sem, bias_sem, out_sem):
    wid = lax.axis_index("s") * NC + lax.axis_index("c")
    base = wid * PER_W

    pltpu.sync_copy(idx_h.at[wid], idx_v)

    for r in range(2 * NCH):
        off = 0 if r < NCH else NUM_ROWS8
        for g in range(NGRP):
            v = idx_v[r, pl.ds(g * NLANE, NLANE)]
            idx8_v[r, pl.ds(g * NLANE, NLANE)] = (
                lax.shift_right_logical(v, 3) + off)

    def issue(j, p):
        return (
            pltpu.async_copy(ue_h.at[idx_v.at[j]], ubuf.at[p], emb_sem.at[p]),
            pltpu.async_copy(be_h.at[idx_v.at[NCH + j]], bbuf.at[p],
                             emb_sem.at[p]),
            pltpu.async_copy(bias_h.at[idx8_v.at[j]], ubias8.at[p],
                             bias_sem.at[p]),
            pltpu.async_copy(bias_h.at[idx8_v.at[NCH + j]], bbias8.at[p],
                             bias_sem.at[p]),
        )

    accs = [jnp.zeros((NLANE,), jnp.float32) for _ in range(2 * NACC)]
    outcps = []
    pend = [issue(0, 0), issue(1, 1), issue(2, 2)]
    for j in range(NCH):
        p = j % NBUF
        cps = pend[0]
        cps[2].wait()
        cps[3].wait()
        for g in range(NGRP):
            pos = lax.iota(jnp.int32, NLANE) + (g * NLANE)
            ui = idx_v[j, pl.ds(g * NLANE, NLANE)]
            bi = idx_v[NCH + j, pl.ds(g * NLANE, NLANE)]
            uv = plsc.load_gather(ubias8.at[p], [pos, jnp.bitwise_and(ui, 7)])
            bv = plsc.load_gather(bbias8.at[p], [pos, jnp.bitwise_and(bi, 7)])
            sbias_f[p, pl.ds(g * NLANE, NLANE)] = uv + bv
        off = base + j * CHUNK
        outcps.append(pltpu.async_copy(
            sbias_f.at[p], sbg_o.at[pl.ds(off, CHUNK)], out_sem))

        cps[0].wait()
        cps[1].wait()
        ub_j, bb_j = ubuf.at[p], bbuf.at[p]

        def row_body(r, accs, ub_j=ub_j, bb_j=bb_j):
            out = []
            for k in range(2):
                for c in range(NACC):
                    u = ub_j[2 * r + k, pl.ds(c * NLANE, NLANE)]
                    b = bb_j[2 * r + k, pl.ds(c * NLANE, NLANE)]
                    out.append(accs[k * NACC + c] + u * b)
            return tuple(out)

        accs = list(lax.fori_loop(0, CHUNK // 2, row_body, tuple(accs)))
        pend.pop(0)
        if j + NBUF < NCH:
            pend.append(issue(j + NBUF, p))

    total = accs[0]
    for c in range(1, 2 * NACC):
        total = total + accs[c]
    acc_v[...] = total
    for cp in outcps:
        cp.wait()
    pltpu.sync_copy(acc_v, partials_o.at[pl.ds(wid * NLANE, NLANE)])


def _tc_body(p_ref, s_ref, o_ref):
    total = jnp.sum(p_ref[...])
    o_ref[...] = jax.nn.sigmoid(total + s_ref[...])


def kernel(inputs, user_embedding, user_bias, book_embedding, book_bias):
    uidx = inputs[:, 0].reshape(NW, NCH, CHUNK)
    bidx = inputs[:, 1].reshape(NW, NCH, CHUNK)
    idx_all = jnp.concatenate([uidx, bidx], axis=1)
    bias_all = jnp.concatenate(
        [user_bias.T, book_bias.T], axis=0).reshape(2 * NUM_ROWS8, 8)

    mesh = plsc.VectorSubcoreMesh(
        core_axis_name="c", subcore_axis_name="s",
        num_cores=NC, num_subcores=NS)

    sc = pl.kernel(
        _sc_body,
        out_type=(
            jax.ShapeDtypeStruct((NW * NLANE,), jnp.float32),
            jax.ShapeDtypeStruct((B,), jnp.float32),
        ),
        mesh=mesh,
        scratch_types=[
            pltpu.VMEM((2 * NCH, CHUNK), jnp.int32),
            pltpu.VMEM((2 * NCH, CHUNK), jnp.int32),
            pltpu.VMEM((NBUF, CHUNK, D), jnp.float32),
            pltpu.VMEM((NBUF, CHUNK, D), jnp.float32),
            pltpu.VMEM((NBUF, CHUNK, 8), jnp.float32),
            pltpu.VMEM((NBUF, CHUNK, 8), jnp.float32),
            pltpu.VMEM((NBUF, CHUNK), jnp.float32),
            pltpu.VMEM((NLANE,), jnp.float32),
            pltpu.SemaphoreType.DMA((NBUF,)),
            pltpu.SemaphoreType.DMA((NBUF,)),
            pltpu.SemaphoreType.DMA,
        ],
        compiler_params=pltpu.CompilerParams(
            use_tc_tiling_on_sc=False, needs_layout_passes=False),
    )
    partials, sbg = sc(idx_all, user_embedding, book_embedding, bias_all)

    out = pl.pallas_call(
        _tc_body,
        out_shape=jax.ShapeDtypeStruct((B // D, D), jnp.float32),
    )(partials.reshape(NW * NLANE // D, D), sbg.reshape(B // D, D))
    return out.reshape(B, 1)
